# baseline (speedup 1.0000x reference)
import functools

import jax
import jax.numpy as jnp
from jax import lax
from jax.experimental import pallas as pl
from jax.experimental.pallas import tpu as pltpu
from jax.experimental.pallas import tpu_sc as plsc

_ROWS = 16384
_NW = 32
_ROWS_PER_W = _ROWS // _NW
_HALF = _ROWS_PER_W // 2
_CH = 8
_NITER = _HALF // _CH
_NBUF = 4
_A = 2
_NG = _NITER // _NBUF


def _sc_copy(x_hbm, o_hbm, tbuf, sbuf_all, tin, tout, sin, sout):
    c = lax.axis_index("c")
    s = lax.axis_index("s")
    sbuf = sbuf_all.at[s]
    base = (s * 2 + c) * _ROWS_PER_W

    def make_path(buf, in_sems, out_sems, off):
        def in_copy(k, b):
            return pltpu.make_async_copy(
                x_hbm.at[pl.ds(off + k * _CH, _CH)], buf.at[b], in_sems.at[b]
            )

        def out_copy(k, b):
            return pltpu.make_async_copy(
                buf.at[b], o_hbm.at[pl.ds(off + k * _CH, _CH)], out_sems.at[b]
            )

        return in_copy, out_copy

    paths = [
        make_path(tbuf, tin, tout, base),
        make_path(sbuf, sin, sout, base + _HALF),
    ]

    def body(g):
        for b in range(_NBUF):
            k = g * _NBUF + b
            for in_copy, out_copy in paths:

                @pl.when(k >= _NBUF)
                def _():
                    out_copy(k - _NBUF, b).wait()

                in_copy(k, b).start()

            bb = (b - _A) % _NBUF
            for in_copy, out_copy in paths:

                @pl.when(k >= _A)
                def _():
                    in_copy(k - _A, bb).wait()
                    out_copy(k - _A, bb).start()

    pl.loop(0, _NG)(body)
    for in_copy, out_copy in paths:
        for k in range(_NITER - _A, _NITER):
            in_copy(k, k % _NBUF).wait()
            out_copy(k, k % _NBUF).start()
        for k in range(_NITER - _NBUF, _NITER):
            out_copy(k, k % _NBUF).wait()


def kernel(inputs, memories):
    del memories
    B, T, d = inputs.shape
    x = inputs.reshape(B * T, d)
    mesh = plsc.VectorSubcoreMesh(core_axis_name="c", subcore_axis_name="s")
    run = functools.partial(
        pl.kernel,
        mesh=mesh,
        out_type=jax.ShapeDtypeStruct((B * T, d), jnp.float32),
        scratch_types=[
            pltpu.VMEM((_NBUF, _CH, d), jnp.float32),
            pltpu.VMEM_SHARED((16, _NBUF, _CH, d), jnp.float32),
            pltpu.SemaphoreType.DMA((_NBUF,)),
            pltpu.SemaphoreType.DMA((_NBUF,)),
            pltpu.SemaphoreType.DMA((_NBUF,)),
            pltpu.SemaphoreType.DMA((_NBUF,)),
        ],
    )(_sc_copy)
    return run(x).reshape(B, T, d)


# speedup vs baseline: 1.0149x; 1.0149x over previous
import functools

import jax
import jax.numpy as jnp
from jax import lax
from jax.experimental import pallas as pl
from jax.experimental.pallas import tpu as pltpu
from jax.experimental.pallas import tpu_sc as plsc

_ROWS = 16384
_D = 2048
_NW = 32
_ROWS_PER_W = _ROWS // _NW
_CH = 8
_NITER = _ROWS_PER_W // _CH
_NBUF = 8
_A = 4
_NG = _NITER // _NBUF


def _sc_copy(x_hbm, o_hbm, sbuf, in_sems, out_sems):
    c = lax.axis_index("c")
    s = lax.axis_index("s")
    buf = sbuf.at[s]
    wid = s * 2 + c
    base = wid * _ROWS_PER_W

    def in_copy(k, b):
        return pltpu.make_async_copy(
            x_hbm.at[pl.ds(base + k * _CH, _CH)], buf.at[b], in_sems.at[b]
        )

    def out_copy(k, b):
        return pltpu.make_async_copy(
            buf.at[b], o_hbm.at[pl.ds(base + k * _CH, _CH)], out_sems.at[b]
        )

    def body(g):
        for b in range(_NBUF):
            k = g * _NBUF + b

            @pl.when(k >= _NBUF)
            def _():
                out_copy(k - _NBUF, b).wait()

            in_copy(k, b).start()

            bb = (b - _A) % _NBUF

            @pl.when(k >= _A)
            def _():
                in_copy(k - _A, bb).wait()
                out_copy(k - _A, bb).start()

    pl.loop(0, _NG)(body)
    for k in range(_NITER - _A, _NITER):
        b = k % _NBUF
        in_copy(k, b).wait()
        out_copy(k, b).start()
    for k in range(_NITER - _NBUF, _NITER):
        out_copy(k, k % _NBUF).wait()


def kernel(inputs, memories):
    del memories
    B, T, d = inputs.shape
    x = inputs.reshape(B * T, d)
    mesh = plsc.VectorSubcoreMesh(core_axis_name="c", subcore_axis_name="s")
    run = functools.partial(
        pl.kernel,
        mesh=mesh,
        out_type=jax.ShapeDtypeStruct((B * T, d), jnp.float32),
        scratch_types=[
            pltpu.VMEM_SHARED((16, _NBUF, _CH, d), jnp.float32),
            pltpu.SemaphoreType.DMA((_NBUF,)),
            pltpu.SemaphoreType.DMA((_NBUF,)),
        ],
    )(_sc_copy)
    return run(x).reshape(B, T, d)


# speedup vs baseline: 1.0217x; 1.0067x over previous
import functools

import jax
import jax.numpy as jnp
from jax import lax
from jax.experimental import pallas as pl
from jax.experimental.pallas import tpu as pltpu
from jax.experimental.pallas import tpu_sc as plsc

_NUM_CORES = 2
_NUM_SUBCORES = 16
_NW = _NUM_CORES * _NUM_SUBCORES
_CH = 8
_NBUF = 4
_A = 2


def _sc_copy_body(n_iter, x_hbm, o_hbm, sbuf_all, in_sems, out_sems):
    c = lax.axis_index("c")
    s = lax.axis_index("s")
    rows_per_w = n_iter * _CH
    base = (s * _NUM_CORES + c) * rows_per_w
    buf = sbuf_all.at[s]

    def in_copy(k, b):
        return pltpu.make_async_copy(
            x_hbm.at[pl.ds(base + k * _CH, _CH)], buf.at[b], in_sems.at[b]
        )

    def out_copy(k, b):
        return pltpu.make_async_copy(
            buf.at[b], o_hbm.at[pl.ds(base + k * _CH, _CH)], out_sems.at[b]
        )

    def body(g):
        for b in range(_NBUF):
            k = g * _NBUF + b

            @pl.when(k >= _NBUF)
            def _():
                out_copy(k - _NBUF, b).wait()

            in_copy(k, b).start()

            bb = (b - _A) % _NBUF

            @pl.when(k >= _A)
            def _():
                in_copy(k - _A, bb).wait()
                out_copy(k - _A, bb).start()

    pl.loop(0, n_iter // _NBUF)(body)
    for k in range(n_iter - _A, n_iter):
        in_copy(k, k % _NBUF).wait()
        out_copy(k, k % _NBUF).start()
    for k in range(n_iter - _NBUF, n_iter):
        out_copy(k, k % _NBUF).wait()


def kernel(inputs, memories):
    del memories
    B, T, d = inputs.shape
    rows = B * T
    n_iter = rows // (_NW * _CH)
    x = inputs.reshape(rows, d)
    mesh = plsc.VectorSubcoreMesh(core_axis_name="c", subcore_axis_name="s")
    run = functools.partial(
        pl.kernel,
        mesh=mesh,
        out_type=jax.ShapeDtypeStruct((rows, d), jnp.float32),
        scratch_types=[
            pltpu.VMEM_SHARED((_NUM_SUBCORES, _NBUF, _CH, d), jnp.float32),
            pltpu.SemaphoreType.DMA((_NBUF,)),
            pltpu.SemaphoreType.DMA((_NBUF,)),
        ],
    )(functools.partial(_sc_copy_body, n_iter))
    return run(x).reshape(B, T, d)
